# baseline (speedup 1.0000x reference)
import functools

import jax
import jax.numpy as jnp
from jax import lax
from jax.experimental import pallas as pl
from jax.experimental.pallas import tpu as pltpu
from jax.experimental.pallas import tpu_sc as plsc

PHONEME_SIZE = 1000
D = 64
BATCH = 16384
SEQ = 50

_INFO = plsc.get_sparse_core_info()
_NC = _INFO.num_cores
_NS = _INFO.num_subcores
_NW = _NC * _NS
_BT = 128
_NBT = BATCH // _BT
_K = _NBT // _NW
_NJ = _K * SEQ
_NBUF = 2


@functools.partial(
    pl.kernel,
    out_type=jax.ShapeDtypeStruct((SEQ, D // 8, _NBT, 8, _BT), jnp.float32),
    mesh=plsc.VectorSubcoreMesh(core_axis_name="c", subcore_axis_name="s"),
    compiler_params=pltpu.CompilerParams(
        use_tc_tiling_on_sc=False, needs_layout_passes=False
    ),
    scratch_types=[
        pltpu.VMEM((_K, SEQ, _BT), jnp.int32),
        pltpu.VMEM((PHONEME_SIZE, D), jnp.float32),
    ]
    + [pltpu.VMEM((D // 8, 8, _BT + 1), jnp.float32)] * _NBUF
    + [pltpu.SemaphoreType.DMA] * _NBUF,
)
def _embed_sc(xt_hbm, table_hbm, out_hbm, idx_v, table_v, *bufs):
    t_v = bufs[:_NBUF]
    sem_s = bufs[_NBUF:]
    wid = lax.axis_index("s") * _NC + lax.axis_index("c")
    bt0 = wid * _K

    pltpu.sync_copy(table_hbm, table_v)
    for k in range(_K):
        pltpu.sync_copy(xt_hbm.at[:, pl.ds((bt0 + k) * _BT, _BT)], idx_v.at[k])

    def split(j):
        return j & (_K - 1), j >> 2

    def writeback(j, b):
        k, s = split(j)
        return pltpu.make_async_copy(
            t_v[b].at[:, :, pl.ds(0, _BT)], out_hbm.at[s, :, bt0 + k], sem_s[b]
        )

    dtvec = [(c * 16 + lax.iota(jnp.int32, 16)) >> 3 for c in range(D // 16)]
    divec = [(c * 16 + lax.iota(jnp.int32, 16)) & 7 for c in range(D // 16)]

    def transpose(j, b):
        k, s = split(j)
        for bg in range(_BT // 16):
            rvec = idx_v[k, s, pl.ds(bg * 16, 16)]
            rs = [rvec[l] for l in range(16)]
            vs = [
                [table_v[rs[l], pl.ds(c * 16, 16)] for c in range(D // 16)]
                for l in range(16)
            ]
            for l in range(16):
                col = jnp.full((16,), bg * 16 + l, jnp.int32)
                for c in range(D // 16):
                    plsc.store_scatter(
                        t_v[b], [dtvec[c], divec[c], col], vs[l][c]
                    )

    def body(t, carry):
        for i in range(_NBUF):
            j = _NBUF * t + i

            @pl.when(t > 0)
            def _():
                writeback(j - _NBUF, i).wait()

            transpose(j, i)
            writeback(j, i).start()

        return carry

    lax.fori_loop(0, _NJ // _NBUF, body, 0)
    for b in range(_NBUF):
        writeback(_NJ - _NBUF + b, b).wait()


def kernel(x, weight):
    xt = x.astype(jnp.int32).T
    p = _embed_sc(xt, weight)
    return p.transpose(2, 4, 0, 1, 3).reshape(BATCH, SEQ, D)


# speedup vs baseline: 1.1621x; 1.1621x over previous
import functools

import jax
import jax.numpy as jnp
from jax import lax
from jax.experimental import pallas as pl
from jax.experimental.pallas import tpu as pltpu
from jax.experimental.pallas import tpu_sc as plsc

PHONEME_SIZE = 1000
D = 64
BATCH = 16384
SEQ = 50

_INFO = plsc.get_sparse_core_info()
_NC = _INFO.num_cores
_NS = _INFO.num_subcores
_NW = _NC * _NS
_BT = 128
_NBT = BATCH // _BT
_K = _NBT // _NW
_NJ = _K * SEQ
_NBUF = 2


@functools.partial(
    pl.kernel,
    out_type=jax.ShapeDtypeStruct((SEQ, D // 8, _NBT, 8, _BT), jnp.float32),
    mesh=plsc.VectorSubcoreMesh(core_axis_name="c", subcore_axis_name="s"),
    compiler_params=pltpu.CompilerParams(
        use_tc_tiling_on_sc=False, needs_layout_passes=False
    ),
    scratch_types=[
        pltpu.VMEM((_K, SEQ, _BT), jnp.int32),
        pltpu.VMEM((PHONEME_SIZE, D), jnp.float32),
    ]
    + [pltpu.VMEM((D // 8, 8, _BT + 1), jnp.float32)] * _NBUF
    + [pltpu.SemaphoreType.DMA] * _NBUF,
)
def _embed_sc(xt_hbm, table_hbm, out_hbm, idx_v, table_v, *bufs):
    t_v = bufs[:_NBUF]
    sem_s = bufs[_NBUF:]
    wid = lax.axis_index("s") * _NC + lax.axis_index("c")
    bt0 = wid * _K

    pltpu.sync_copy(table_hbm, table_v)
    for k in range(_K):
        pltpu.sync_copy(xt_hbm.at[:, pl.ds((bt0 + k) * _BT, _BT)], idx_v.at[k])

    def split(j):
        return j & (_K - 1), j >> 2

    def writeback(j, b):
        k, s = split(j)
        return pltpu.make_async_copy(
            t_v[b].at[:, :, pl.ds(0, _BT)], out_hbm.at[s, :, bt0 + k], sem_s[b]
        )

    dtvec = [(c * 16 + lax.iota(jnp.int32, 16)) >> 3 for c in range(D // 16)]
    divec = [(c * 16 + lax.iota(jnp.int32, 16)) & 7 for c in range(D // 16)]

    def transpose(j, b):
        k, s = split(j)
        for bg in range(_BT // 16):
            rvec = idx_v[k, s, pl.ds(bg * 16, 16)]
            prev = None
            for l in range(16):
                r = rvec[l]
                vs = [table_v[r, pl.ds(c * 16, 16)] for c in range(D // 16)]
                if prev is not None:
                    pbi, pvs = prev
                    col = jnp.full((16,), pbi, jnp.int32)
                    for c in range(D // 16):
                        plsc.store_scatter(
                            t_v[b], [dtvec[c], divec[c], col], pvs[c]
                        )
                prev = (bg * 16 + l, vs)
            pbi, pvs = prev
            col = jnp.full((16,), pbi, jnp.int32)
            for c in range(D // 16):
                plsc.store_scatter(t_v[b], [dtvec[c], divec[c], col], pvs[c])

    def body(t, carry):
        for i in range(_NBUF):
            j = _NBUF * t + i

            @pl.when(t > 0)
            def _():
                writeback(j - _NBUF, i).wait()

            transpose(j, i)
            writeback(j, i).start()

        return carry

    lax.fori_loop(0, _NJ // _NBUF, body, 0)
    for b in range(_NBUF):
        writeback(_NJ - _NBUF + b, b).wait()


def kernel(x, weight):
    xt = x.astype(jnp.int32).T
    p = _embed_sc(xt, weight)
    return p.transpose(2, 4, 0, 1, 3).reshape(BATCH, SEQ, D)
